# baseline (speedup 1.0000x reference)
import functools

import jax
import jax.numpy as jnp
from jax import lax
from jax.experimental import pallas as pl
from jax.experimental.pallas import tpu as pltpu
from jax.experimental.pallas import tpu_sc as plsc

NC = 2
NS = 16
NW = NC * NS

S = 50
NB = 16384
B = S * NB
D = 64
BLK = 128
NBLK = B // BLK
BPW = B // NW
GPW = BPW // BLK
JB = NB // BLK

_mesh = plsc.VectorSubcoreMesh(
    core_axis_name="c", subcore_axis_name="s", num_cores=NC, num_subcores=NS
)


@functools.partial(
    pl.kernel,
    out_type=jax.ShapeDtypeStruct((S, D // 8, JB, 8, BLK), jnp.float32),
    mesh=_mesh,
    compiler_params=pltpu.CompilerParams(
        use_tc_tiling_on_sc=False, needs_layout_passes=False),
    scratch_types=[
        pltpu.VMEM((BPW,), jnp.int32),
        pltpu.VMEM((BLK, D), jnp.float32),
        pltpu.VMEM((BLK, D), jnp.float32),
        pltpu.VMEM((D, BLK), jnp.float32),
        pltpu.VMEM((D, BLK), jnp.float32),
        pltpu.SemaphoreType.DMA,
        pltpu.SemaphoreType.DMA,
        pltpu.SemaphoreType.DMA,
        pltpu.SemaphoreType.DMA,
    ],
)
def _embed_gather(idx_hbm, table_hbm, out_hbm,
                  idx_v, gbuf0, gbuf1, tbuf0, tbuf1, g0, g1, o0, o1):
    wid = lax.axis_index("s") * NC + lax.axis_index("c")
    base = wid * BPW
    blk0 = wid * GPW

    pltpu.sync_copy(idx_hbm.at[pl.ds(base, BPW)], idx_v)

    iota = lax.iota(jnp.int32, 16)
    brows = [16 * bg + iota for bg in range(8)]

    def fire(g, buf, sem):
        off = pl.multiple_of(g * BLK, BLK)
        pltpu.async_copy(table_hbm.at[idx_v.at[pl.ds(off, BLK)]], buf, sem)

    def drain_g(buf, sem):
        pltpu.make_async_copy(table_hbm.at[pl.ds(0, BLK)], buf, sem).wait()

    def transpose(gbuf, tbuf):
        def trow(i, _):
            for k in range(4):
                d = 4 * i + k
                col = jnp.full((16,), d, jnp.int32)
                for bg in range(8):
                    v = plsc.load_gather(gbuf, [brows[bg], col])
                    tbuf[d, pl.ds(16 * bg, 16)] = v
            return 0
        lax.fori_loop(0, D // 4, trow, 0)

    def write(blk, tbuf, sem):
        s = blk // JB
        j = lax.rem(blk, JB)
        for gr in range(D // 8):
            pltpu.async_copy(
                tbuf.at[pl.ds(8 * gr, 8)], out_hbm.at[s, gr, j], sem)

    def drain_o(tbuf, sem):
        for gr in range(D // 8):
            pltpu.make_async_copy(
                out_hbm.at[0, 0, 0], tbuf.at[pl.ds(8 * gr, 8)], sem).wait()

    fire(0, gbuf0, g0)
    fire(1, gbuf1, g1)
    write(blk0, tbuf0, o0)
    write(blk0 + 1, tbuf1, o1)

    def body(i, _):
        ga = 2 * i
        drain_g(gbuf0, g0)
        drain_o(tbuf0, o0)
        transpose(gbuf0, tbuf0)
        fire(jnp.minimum(ga + 2, GPW - 1), gbuf0, g0)
        write(blk0 + ga, tbuf0, o0)
        drain_g(gbuf1, g1)
        drain_o(tbuf1, o1)
        transpose(gbuf1, tbuf1)
        fire(jnp.minimum(ga + 3, GPW - 1), gbuf1, g1)
        write(blk0 + ga + 1, tbuf1, o1)
        return 0

    lax.fori_loop(0, GPW // 2, body, 0)
    drain_g(gbuf0, g0)
    drain_g(gbuf1, g1)
    drain_o(tbuf0, o0)
    drain_o(tbuf1, o1)


def kernel(x, table):
    flat = x.T.reshape(-1).astype(jnp.int32)
    out5 = _embed_gather(flat, table)
    return out5.transpose(2, 4, 0, 1, 3).reshape(NB, S, D)


# speedup vs baseline: 1.6986x; 1.6986x over previous
import functools

import jax
import jax.numpy as jnp
from jax import lax
from jax.experimental import pallas as pl
from jax.experimental.pallas import tpu as pltpu
from jax.experimental.pallas import tpu_sc as plsc

NC = 2
NS = 16
NW = NC * NS

B = 16384 * 50
D = 64
NUM_ROWS = 1000000
BPW = B // NW
C = 512
NG = BPW // C

_mesh = plsc.VectorSubcoreMesh(
    core_axis_name="c", subcore_axis_name="s", num_cores=NC, num_subcores=NS
)


@functools.partial(
    pl.kernel,
    out_type=jax.ShapeDtypeStruct((B, D), jnp.float32),
    mesh=_mesh,
    compiler_params=pltpu.CompilerParams(use_tc_tiling_on_sc=False),
    scratch_types=[
        pltpu.VMEM((BPW,), jnp.int32),
        pltpu.VMEM((C, D), jnp.float32),
        pltpu.VMEM((C, D), jnp.float32),
        pltpu.SemaphoreType.DMA,
        pltpu.SemaphoreType.DMA,
    ],
)
def _embed_gather(idx_hbm, table_hbm, out_hbm, idx_v, buf0, buf1, g0, g1):
    wid = lax.axis_index("s") * NC + lax.axis_index("c")
    base = wid * BPW

    pltpu.sync_copy(idx_hbm.at[pl.ds(base, BPW)], idx_v)

    def remap(i, _):
        o = pl.multiple_of(i * 16, 16)
        v = idx_v[pl.ds(o, 16)]
        q = lax.bitwise_and(v, 2047)
        tl = lax.shift_right_logical(999423 - v, 31)
        ht = lax.shift_right_logical(287 - q, 31)
        hm = lax.shift_right_logical(q, 10)
        h = hm + tl * (ht - hm)
        p = q - h * (1024 - 736 * tl)
        idx_v[pl.ds(o, 16)] = (v - q) + 2 * p + h
        return 0

    lax.fori_loop(0, BPW // 16, remap, 0)

    def fire(group, buf, sem):
        off = pl.multiple_of(group * C, C)
        pltpu.async_copy(table_hbm.at[idx_v.at[pl.ds(off, C)]], buf, sem)

    def drain(buf, sem):
        pltpu.make_async_copy(table_hbm.at[pl.ds(0, C)], buf, sem).wait()

    def write(group, buf):
        row = base + pl.multiple_of(group * C, C)
        pltpu.sync_copy(buf, out_hbm.at[pl.ds(row, C)])

    fire(0, buf0, g0)

    def body(i, _):
        ga = 2 * i
        fire(ga + 1, buf1, g1)
        drain(buf0, g0)
        write(ga, buf0)
        gb = jnp.minimum(ga + 2, 2 * (NG // 2) - 2)
        fire(gb, buf0, g0)
        drain(buf1, g1)
        write(ga + 1, buf1)
        return 0

    lax.fori_loop(0, NG // 2, body, 0)
    drain(buf0, g0)


_TCOLS = 2048


_NFULL = NUM_ROWS // _TCOLS
_TAIL = NUM_ROWS - _NFULL * _TCOLS


def _detile_body(in_ref, out_ref):
    x = in_ref[...]
    xt = x.T
    last = pl.program_id(0) == _NFULL

    @pl.when(jnp.logical_not(last))
    def _():
        out_ref[:, 0:D] = xt[0:_TCOLS // 2, :]
        out_ref[:, D:2 * D] = xt[_TCOLS // 2:_TCOLS, :]

    @pl.when(last)
    def _():
        out_ref[0:_TAIL // 2, 0:D] = xt[0:_TAIL // 2, :]
        out_ref[0:_TAIL // 2, D:2 * D] = xt[_TAIL // 2:_TAIL, :]


_detile_table = pl.pallas_call(
    _detile_body,
    out_shape=jax.ShapeDtypeStruct((NUM_ROWS // 2, 2 * D), jnp.float32),
    grid=(pl.cdiv(NUM_ROWS, _TCOLS),),
    in_specs=[pl.BlockSpec((D, _TCOLS), lambda i: (0, i))],
    out_specs=pl.BlockSpec((_TCOLS // 2, 2 * D), lambda i: (i, 0)),
)


def kernel(x, table):
    tbl = _detile_table(table.T).reshape(NUM_ROWS, D)
    flat = x.T.reshape(-1).astype(jnp.int32)
    out = _embed_gather(flat, tbl)
    s, b = x.shape[1], x.shape[0]
    return out.reshape(s, b, D).transpose(1, 0, 2)


# speedup vs baseline: 2.3965x; 1.4108x over previous
import functools

import jax
import jax.numpy as jnp
from jax import lax
from jax.experimental import pallas as pl
from jax.experimental.pallas import tpu as pltpu
from jax.experimental.pallas import tpu_sc as plsc

NC = 2
NS = 16
NW = NC * NS

B = 16384 * 50
D = 64
NUM_ROWS = 1000000
BPW = B // NW
C = 512
NG = BPW // C

_mesh = plsc.VectorSubcoreMesh(
    core_axis_name="c", subcore_axis_name="s", num_cores=NC, num_subcores=NS
)


@functools.partial(
    pl.kernel,
    out_type=jax.ShapeDtypeStruct((B // 2, 2 * D), jnp.float32),
    mesh=_mesh,
    compiler_params=pltpu.CompilerParams(use_tc_tiling_on_sc=False),
    scratch_types=[
        pltpu.VMEM((BPW,), jnp.int32),
        pltpu.VMEM((C, D), jnp.float32),
        pltpu.VMEM((C, D), jnp.float32),
        pltpu.SemaphoreType.DMA,
        pltpu.SemaphoreType.DMA,
    ],
)
def _embed_gather(idx_hbm, table_hbm, out_hbm, idx_v, buf0, buf1, g0, g1):
    wid = lax.axis_index("s") * NC + lax.axis_index("c")
    base = wid * BPW

    pltpu.sync_copy(idx_hbm.at[pl.ds(base, BPW)], idx_v)

    def remap(i, _):
        o = pl.multiple_of(i * 16, 16)
        v = idx_v[pl.ds(o, 16)]
        q = lax.bitwise_and(v, 2047)
        tl = lax.shift_right_logical(999423 - v, 31)
        ht = lax.shift_right_logical(287 - q, 31)
        hm = lax.shift_right_logical(q, 10)
        h = hm + tl * (ht - hm)
        p = q - h * (1024 - 736 * tl)
        idx_v[pl.ds(o, 16)] = (v - q) + 2 * p + h
        return 0

    lax.fori_loop(0, BPW // 16, remap, 0)

    def fire(group, buf, sem):
        off = pl.multiple_of(group * C, C)
        pltpu.async_copy(table_hbm.at[idx_v.at[pl.ds(off, C)]], buf, sem)

    def drain(buf, sem):
        pltpu.make_async_copy(table_hbm.at[pl.ds(0, C)], buf, sem).wait()

    def write(group, buf):
        k0 = base + pl.multiple_of(group * C, C)
        s = k0 // 16384
        b0 = lax.rem(k0, 16384)
        h = b0 // 8192
        p0 = s * 8192 + lax.rem(b0, 8192)
        pltpu.sync_copy(
            buf, out_hbm.at[pl.ds(pl.multiple_of(p0, C), C),
                            pl.ds(pl.multiple_of(h * D, D), D)])

    fire(0, buf0, g0)

    def body(i, _):
        ga = 2 * i
        fire(ga + 1, buf1, g1)
        drain(buf0, g0)
        write(ga, buf0)
        gb = jnp.minimum(ga + 2, 2 * (NG // 2) - 2)
        fire(gb, buf0, g0)
        drain(buf1, g1)
        write(ga + 1, buf1)
        return 0

    lax.fori_loop(0, NG // 2, body, 0)
    drain(buf0, g0)


_TCOLS = 2048


_NFULL = NUM_ROWS // _TCOLS
_TAIL = NUM_ROWS - _NFULL * _TCOLS


def _detile_body(in_ref, out_ref):
    x = in_ref[...]
    xt = x.T
    last = pl.program_id(0) == _NFULL

    @pl.when(jnp.logical_not(last))
    def _():
        out_ref[:, 0:D] = xt[0:_TCOLS // 2, :]
        out_ref[:, D:2 * D] = xt[_TCOLS // 2:_TCOLS, :]

    @pl.when(last)
    def _():
        out_ref[0:_TAIL // 2, 0:D] = xt[0:_TAIL // 2, :]
        out_ref[0:_TAIL // 2, D:2 * D] = xt[_TAIL // 2:_TAIL, :]


_detile_table = pl.pallas_call(
    _detile_body,
    out_shape=jax.ShapeDtypeStruct((NUM_ROWS // 2, 2 * D), jnp.float32),
    grid=(pl.cdiv(NUM_ROWS, _TCOLS),),
    in_specs=[pl.BlockSpec((D, _TCOLS), lambda i: (0, i))],
    out_specs=pl.BlockSpec((_TCOLS // 2, 2 * D), lambda i: (i, 0)),
)


def _xpose_body(in_ref, out_ref):
    for j in range(64):
        xt = in_ref[pl.ds(128 * j, 128), :].T
        for g in range(8):
            out_ref[0, g, j, :, :] = xt[8 * g:8 * g + 8, :]
            out_ref[0, g, 64 + j, :, :] = xt[64 + 8 * g:72 + 8 * g, :]


_xform_out = pl.pallas_call(
    _xpose_body,
    out_shape=jax.ShapeDtypeStruct((50, 8, 128, 8, 128), jnp.float32),
    grid=(50,),
    in_specs=[pl.BlockSpec((8192, 2 * D), lambda i: (i, 0))],
    out_specs=pl.BlockSpec((1, 8, 128, 8, 128), lambda i: (i, 0, 0, 0, 0)),
)


def kernel(x, table):
    tbl = _detile_table(table.T).reshape(NUM_ROWS, D)
    flat = x.T.reshape(-1).astype(jnp.int32)
    rm2 = _embed_gather(flat, tbl)
    out5 = _xform_out(rm2)
    return out5.transpose(2, 4, 0, 1, 3).reshape(x.shape[0], x.shape[1], D)
